# baseline (speedup 1.0000x reference)
import functools
import jax
import jax.numpy as jnp
from jax import lax
from jax.experimental import pallas as pl
from jax.experimental.pallas import tpu as pltpu
from jax.experimental.pallas import tpu_sc as plsc

B, N, E = 8, 128, 512
IN_F, HID_F, OUT_F, MSG_F, BOND_F = 64, 128, 64, 64, 16
PAD = 128

NC, NS = 2, 16
NW = NC * NS
ROWS = B * E
RPW = ROWS // NW

_sc_mesh = plsc.VectorSubcoreMesh(
    core_axis_name="c", subcore_axis_name="s", num_cores=NC, num_subcores=NS)


def _leaky(x):
    return jnp.where(x >= 0, x, 0.01 * x)


@functools.partial(
    pl.kernel, mesh=_sc_mesh,
    out_type=jax.ShapeDtypeStruct((ROWS, PAD), jnp.float32),
    scratch_types=[pltpu.VMEM((RPW,), jnp.int32),
                   pltpu.VMEM((RPW,), jnp.int32),
                   pltpu.VMEM((RPW, PAD), jnp.float32),
                   pltpu.VMEM((RPW, PAD), jnp.float32),
                   pltpu.SemaphoreType.DMA,
                   pltpu.SemaphoreType.DMA],
)
def _sc_gather(table_hbm, idx1_hbm, idx2_hbm, out_v, idx1_v, idx2_v,
               rows1_v, rows2_v, sem1, sem2):
    wid = lax.axis_index("s") * NC + lax.axis_index("c")
    r0 = wid * RPW
    b = r0 // E
    e0 = r0 % E
    boff = b * N
    l1 = pltpu.async_copy(idx1_hbm.at[pl.ds(e0, RPW)], idx1_v, sem1)
    l2 = pltpu.async_copy(idx2_hbm.at[pl.ds(e0, RPW)], idx2_v, sem2)
    l1.wait()
    l2.wait()
    for i in range(RPW // 16):
        sl = pl.ds(i * 16, 16)
        idx1_v[sl] = idx1_v[sl] + boff
        idx2_v[sl] = idx2_v[sl] + boff
    g1 = pltpu.async_copy(table_hbm.at[idx1_v], rows1_v, sem1)
    g2 = pltpu.async_copy(table_hbm.at[idx2_v], rows2_v, sem2)
    g1.wait()
    g2.wait()
    def _merge_row(i, _):
        for j in range(IN_F // 16):
            sl = pl.ds(IN_F + j * 16, 16)
            rows1_v[i, sl] = rows2_v[i, sl]
        return 0
    lax.fori_loop(0, RPW, _merge_row, 0, unroll=4)
    pltpu.sync_copy(rows1_v, out_v.at[pl.ds(r0, RPW)])


def _tc_kernel(vec_ref, bonds_ref, oh2_ref, sites_ref,
               W_self_ref, W_pool_ref, b_eq_ref, att_W_ref, att_b_ref,
               W1_ref, b1_ref, W2_ref, b2_ref, out_ref):
    W_eff = W_self_ref[...] + W_pool_ref[...] * (1.0 / N)
    lat = (jnp.dot(vec_ref[...], W_eff[:2 * IN_F], preferred_element_type=jnp.float32)
           + jnp.dot(bonds_ref[...], W_eff[2 * IN_F:], preferred_element_type=jnp.float32)
           + b_eq_ref[...])
    lat = _leaky(lat)
    logits = jnp.sum(lat * att_W_ref[...].T, axis=1, keepdims=True) + att_b_ref[...]
    lat = lat * jax.nn.sigmoid(logits)

    oh2 = oh2_ref[...]
    sites = sites_ref[...]
    h1 = jnp.dot(sites, W1_ref[:IN_F], preferred_element_type=jnp.float32)
    msgs = []
    for b in range(B):
        msgs.append(jnp.dot(oh2.T, lat[b * E:(b + 1) * E],
                            preferred_element_type=jnp.float32))
    msg = jnp.concatenate(msgs, axis=0)

    v = _leaky(h1 + jnp.dot(msg, W1_ref[IN_F:], preferred_element_type=jnp.float32)
               + b1_ref[...])
    v = _leaky(jnp.dot(v, W2_ref[...], preferred_element_type=jnp.float32)
               + b2_ref[...])
    out_ref[...] = sites + v


def kernel(sites, bonds, idx1, idx2, idx2_oh, W_self, W_pool, b_eq, att_W, att_b, W1, b1, W2, b2):
    C = 2 * IN_F + BOND_F
    sites_flat = sites.reshape(B * N, IN_F)
    bonds_flat = bonds.reshape(B * E, BOND_F)
    table = jnp.concatenate([sites_flat, sites_flat], axis=1)

    vec = _sc_gather(table, idx1, idx2)

    fixed2 = lambda: (0, 0)
    sites_out = pl.pallas_call(
        _tc_kernel,
        in_specs=[pl.BlockSpec((ROWS, PAD), fixed2),
                  pl.BlockSpec((ROWS, BOND_F), fixed2),
                  pl.BlockSpec((E, N), fixed2),
                  pl.BlockSpec((B * N, IN_F), fixed2),
                  pl.BlockSpec((C, MSG_F), fixed2),
                  pl.BlockSpec((C, MSG_F), fixed2),
                  pl.BlockSpec((1, MSG_F), fixed2),
                  pl.BlockSpec((MSG_F, 1), fixed2),
                  pl.BlockSpec((1, 1), fixed2),
                  pl.BlockSpec((IN_F + MSG_F, HID_F), fixed2),
                  pl.BlockSpec((1, HID_F), fixed2),
                  pl.BlockSpec((HID_F, OUT_F), fixed2),
                  pl.BlockSpec((1, OUT_F), fixed2)],
        out_specs=pl.BlockSpec((B * N, OUT_F), fixed2),
        out_shape=jax.ShapeDtypeStruct((B * N, OUT_F), jnp.float32),
    )(vec, bonds_flat, idx2_oh, sites_flat, W_self, W_pool,
      b_eq.reshape(1, MSG_F), att_W, att_b.reshape(1, 1),
      W1, b1.reshape(1, HID_F), W2, b2.reshape(1, OUT_F))

    return (sites_out.reshape(B, N, OUT_F), bonds)


# speedup vs baseline: 1.0084x; 1.0084x over previous
import functools
import jax
import jax.numpy as jnp
from jax import lax
from jax.experimental import pallas as pl
from jax.experimental.pallas import tpu as pltpu
from jax.experimental.pallas import tpu_sc as plsc

B, N, E = 8, 128, 512
IN_F, HID_F, OUT_F, MSG_F, BOND_F = 64, 128, 64, 64, 16
PAD = 128

NC, NS = 2, 16
NW = NC * NS
ROWS = B * E
RPW = ROWS // NW

_sc_mesh = plsc.VectorSubcoreMesh(
    core_axis_name="c", subcore_axis_name="s", num_cores=NC, num_subcores=NS)


def _leaky(x):
    return jnp.where(x >= 0, x, 0.01 * x)


@functools.partial(
    pl.kernel, mesh=_sc_mesh,
    out_type=(jax.ShapeDtypeStruct((ROWS, PAD), jnp.float32),
              jax.ShapeDtypeStruct((ROWS, PAD), jnp.float32)),
    scratch_types=[pltpu.VMEM((RPW,), jnp.int32),
                   pltpu.VMEM((RPW,), jnp.int32),
                   pltpu.VMEM((RPW, PAD), jnp.float32),
                   pltpu.VMEM((RPW, PAD), jnp.float32),
                   pltpu.SemaphoreType.DMA,
                   pltpu.SemaphoreType.DMA],
)
def _sc_gather(table_hbm, idx1_hbm, idx2_hbm, out_s, out_r,
               idx1_v, idx2_v, rows1_v, rows2_v, sem1, sem2):
    wid = lax.axis_index("s") * NC + lax.axis_index("c")
    r0 = wid * RPW
    b = r0 // E
    e0 = r0 % E
    boff = b * N
    l1 = pltpu.async_copy(idx1_hbm.at[pl.ds(e0, RPW)], idx1_v, sem1)
    l2 = pltpu.async_copy(idx2_hbm.at[pl.ds(e0, RPW)], idx2_v, sem2)
    l1.wait()
    l2.wait()
    for i in range(RPW // 16):
        sl = pl.ds(i * 16, 16)
        idx1_v[sl] = idx1_v[sl] + boff
        idx2_v[sl] = idx2_v[sl] + boff
    g1 = pltpu.async_copy(table_hbm.at[idx1_v], rows1_v, sem1)
    g2 = pltpu.async_copy(table_hbm.at[idx2_v], rows2_v, sem2)
    g1.wait()
    w1 = pltpu.async_copy(rows1_v, out_s.at[pl.ds(r0, RPW)], sem1)
    g2.wait()
    w2 = pltpu.async_copy(rows2_v, out_r.at[pl.ds(r0, RPW)], sem2)
    w1.wait()
    w2.wait()


def _tc_kernel(ss_ref, sr_ref, bonds_ref, oh2_ref, sites_ref,
               W_self_ref, W_pool_ref, b_eq_ref, att_W_ref, att_b_ref,
               W1_ref, b1_ref, W2_ref, b2_ref, out_ref):
    W_eff = W_self_ref[...] + W_pool_ref[...] * (1.0 / N)
    ss = ss_ref[0][:, :IN_F]
    sr = sr_ref[0][:, :IN_F]
    lat = (jnp.dot(ss, W_eff[:IN_F], preferred_element_type=jnp.float32)
           + jnp.dot(sr, W_eff[IN_F:2 * IN_F], preferred_element_type=jnp.float32)
           + jnp.dot(bonds_ref[0], W_eff[2 * IN_F:], preferred_element_type=jnp.float32)
           + b_eq_ref[...])
    lat = _leaky(lat)
    logits = jnp.sum(lat * att_W_ref[...].T, axis=1, keepdims=True) + att_b_ref[...]
    lat = lat * jax.nn.sigmoid(logits)

    sites = sites_ref[0]
    msg = jnp.dot(oh2_ref[...].T, lat, preferred_element_type=jnp.float32)

    v = _leaky(jnp.dot(sites, W1_ref[:IN_F], preferred_element_type=jnp.float32)
               + jnp.dot(msg, W1_ref[IN_F:], preferred_element_type=jnp.float32)
               + b1_ref[...])
    v = _leaky(jnp.dot(v, W2_ref[...], preferred_element_type=jnp.float32)
               + b2_ref[...])
    out_ref[0] = sites + v


def kernel(sites, bonds, idx1, idx2, idx2_oh, W_self, W_pool, b_eq, att_W, att_b, W1, b1, W2, b2):
    C = 2 * IN_F + BOND_F
    sites_flat = sites.reshape(B * N, IN_F)
    bonds_flat = bonds.reshape(B * E, BOND_F)
    table = jnp.concatenate(
        [sites_flat, jnp.zeros((B * N, PAD - IN_F), jnp.float32)], axis=1)

    ss, sr = _sc_gather(table, idx1, idx2)

    fixed2 = lambda b: (0, 0)
    batch3 = lambda b: (b, 0, 0)
    sites_out = pl.pallas_call(
        _tc_kernel,
        grid=(B,),
        in_specs=[pl.BlockSpec((1, E, PAD), batch3),
                  pl.BlockSpec((1, E, PAD), batch3),
                  pl.BlockSpec((1, E, BOND_F), batch3),
                  pl.BlockSpec((E, N), fixed2),
                  pl.BlockSpec((1, N, IN_F), batch3),
                  pl.BlockSpec((C, MSG_F), fixed2),
                  pl.BlockSpec((C, MSG_F), fixed2),
                  pl.BlockSpec((1, MSG_F), fixed2),
                  pl.BlockSpec((MSG_F, 1), fixed2),
                  pl.BlockSpec((1, 1), fixed2),
                  pl.BlockSpec((IN_F + MSG_F, HID_F), fixed2),
                  pl.BlockSpec((1, HID_F), fixed2),
                  pl.BlockSpec((HID_F, OUT_F), fixed2),
                  pl.BlockSpec((1, OUT_F), fixed2)],
        out_specs=pl.BlockSpec((1, N, OUT_F), batch3),
        out_shape=jax.ShapeDtypeStruct((B, N, OUT_F), jnp.float32),
    )(ss.reshape(B, E, PAD), sr.reshape(B, E, PAD), bonds, idx2_oh,
      sites, W_self, W_pool,
      b_eq.reshape(1, MSG_F), att_W, att_b.reshape(1, 1),
      W1, b1.reshape(1, HID_F), W2, b2.reshape(1, OUT_F))

    return (sites_out, bonds)


# speedup vs baseline: 1.0745x; 1.0656x over previous
import functools
import jax
import jax.numpy as jnp
from jax import lax
from jax.experimental import pallas as pl
from jax.experimental.pallas import tpu as pltpu
from jax.experimental.pallas import tpu_sc as plsc

B, N, E = 8, 128, 512
IN_F, HID_F, OUT_F, MSG_F, BOND_F = 64, 128, 64, 64, 16
PAD = 128

NC, NS = 2, 16
NW = NC * NS
ROWS = B * E
RPW = ROWS // NW

_sc_mesh = plsc.VectorSubcoreMesh(
    core_axis_name="c", subcore_axis_name="s", num_cores=NC, num_subcores=NS)


def _leaky(x):
    return jnp.where(x >= 0, x, 0.01 * x)


@functools.partial(
    pl.kernel, mesh=_sc_mesh,
    out_type=jax.ShapeDtypeStruct((ROWS, PAD), jnp.float32),
    scratch_types=[pltpu.VMEM((RPW,), jnp.int32),
                   pltpu.VMEM((RPW, PAD), jnp.float32),
                   pltpu.SemaphoreType.DMA],
)
def _sc_gather(table_hbm, idx1_hbm, out_s, idx1_v, rows1_v, sem1):
    wid = lax.axis_index("s") * NC + lax.axis_index("c")
    r0 = wid * RPW
    b = r0 // E
    e0 = r0 % E
    boff = b * N
    pltpu.sync_copy(idx1_hbm.at[pl.ds(e0, RPW)], idx1_v)
    for i in range(RPW // 16):
        sl = pl.ds(i * 16, 16)
        idx1_v[sl] = idx1_v[sl] + boff
    pltpu.async_copy(table_hbm.at[idx1_v], rows1_v, sem1).wait()
    pltpu.sync_copy(rows1_v, out_s.at[pl.ds(r0, RPW)])


def _tc_kernel(ss_ref, bonds_ref, oh2_ref, sites_ref,
               W_self_ref, W_pool_ref, b_eq_ref, att_W_ref, att_b_ref,
               W1_ref, b1_ref, W2_ref, b2_ref, out_ref):
    W_eff = W_self_ref[...] + W_pool_ref[...] * (1.0 / N)
    ss = ss_ref[...][:, :IN_F]
    oh2 = oh2_ref[...]
    sites = sites_ref[...]
    P = jnp.dot(sites, W_eff[IN_F:2 * IN_F], preferred_element_type=jnp.float32)
    recv = jnp.concatenate(
        [jnp.dot(oh2, P[b * N:(b + 1) * N], preferred_element_type=jnp.float32)
         for b in range(B)], axis=0)
    lat = (jnp.dot(ss, W_eff[:IN_F], preferred_element_type=jnp.float32)
           + recv
           + jnp.dot(bonds_ref[...], W_eff[2 * IN_F:], preferred_element_type=jnp.float32)
           + b_eq_ref[...])
    lat = _leaky(lat)
    logits = jnp.sum(lat * att_W_ref[...].T, axis=1, keepdims=True) + att_b_ref[...]
    lat = lat * jax.nn.sigmoid(logits)

    h1 = jnp.dot(sites, W1_ref[:IN_F], preferred_element_type=jnp.float32)
    msgs = []
    for b in range(B):
        msgs.append(jnp.dot(oh2.T, lat[b * E:(b + 1) * E],
                            preferred_element_type=jnp.float32))
    msg = jnp.concatenate(msgs, axis=0)

    v = _leaky(h1 + jnp.dot(msg, W1_ref[IN_F:], preferred_element_type=jnp.float32)
               + b1_ref[...])
    v = _leaky(jnp.dot(v, W2_ref[...], preferred_element_type=jnp.float32)
               + b2_ref[...])
    out_ref[...] = sites + v


def kernel(sites, bonds, idx1, idx2, idx2_oh, W_self, W_pool, b_eq, att_W, att_b, W1, b1, W2, b2):
    C = 2 * IN_F + BOND_F
    sites_flat = sites.reshape(B * N, IN_F)
    bonds_flat = bonds.reshape(B * E, BOND_F)
    table = jnp.concatenate(
        [sites_flat, jnp.zeros((B * N, PAD - IN_F), jnp.float32)], axis=1)

    ss = _sc_gather(table, idx1)

    fixed2 = lambda: (0, 0)
    sites_out = pl.pallas_call(
        _tc_kernel,
        in_specs=[pl.BlockSpec((ROWS, PAD), fixed2),
                  pl.BlockSpec((ROWS, BOND_F), fixed2),
                  pl.BlockSpec((E, N), fixed2),
                  pl.BlockSpec((B * N, IN_F), fixed2),
                  pl.BlockSpec((C, MSG_F), fixed2),
                  pl.BlockSpec((C, MSG_F), fixed2),
                  pl.BlockSpec((1, MSG_F), fixed2),
                  pl.BlockSpec((MSG_F, 1), fixed2),
                  pl.BlockSpec((1, 1), fixed2),
                  pl.BlockSpec((IN_F + MSG_F, HID_F), fixed2),
                  pl.BlockSpec((1, HID_F), fixed2),
                  pl.BlockSpec((HID_F, OUT_F), fixed2),
                  pl.BlockSpec((1, OUT_F), fixed2)],
        out_specs=pl.BlockSpec((B * N, OUT_F), fixed2),
        out_shape=jax.ShapeDtypeStruct((B * N, OUT_F), jnp.float32),
    )(ss, bonds_flat, idx2_oh, sites_flat, W_self, W_pool,
      b_eq.reshape(1, MSG_F), att_W, att_b.reshape(1, 1),
      W1, b1.reshape(1, HID_F), W2, b2.reshape(1, OUT_F))

    return (sites_out.reshape(B, N, OUT_F), bonds)


# speedup vs baseline: 1.0993x; 1.0230x over previous
import functools
import jax
import jax.numpy as jnp
from jax import lax
from jax.experimental import pallas as pl
from jax.experimental.pallas import tpu as pltpu
from jax.experimental.pallas import tpu_sc as plsc

B, N, E = 8, 128, 512
IN_F, HID_F, OUT_F, MSG_F, BOND_F = 64, 128, 64, 64, 16
PAD = 128

NC, NS = 2, 16
NW = NC * NS
ROWS = B * E
RPW = ROWS // NW

_sc_mesh = plsc.VectorSubcoreMesh(
    core_axis_name="c", subcore_axis_name="s", num_cores=NC, num_subcores=NS)


def _leaky(x):
    return jnp.where(x >= 0, x, 0.01 * x)


@functools.partial(
    pl.kernel, mesh=_sc_mesh,
    out_type=(jax.ShapeDtypeStruct((ROWS, PAD), jnp.float32),
              jax.ShapeDtypeStruct((ROWS, PAD), jnp.float32)),
    scratch_types=[pltpu.VMEM((RPW,), jnp.int32),
                   pltpu.VMEM((RPW,), jnp.int32),
                   pltpu.VMEM((RPW, PAD), jnp.float32),
                   pltpu.VMEM((RPW, PAD), jnp.float32),
                   pltpu.SemaphoreType.DMA,
                   pltpu.SemaphoreType.DMA],
)
def _sc_gather(table_hbm, idx1_hbm, idx2_hbm, out_s, out_r,
               idx1_v, idx2_v, rows1_v, rows2_v, sem1, sem2):
    wid = lax.axis_index("s") * NC + lax.axis_index("c")
    r0 = wid * RPW
    b = r0 // E
    e0 = r0 % E
    boff = b * N
    l1 = pltpu.async_copy(idx1_hbm.at[pl.ds(e0, RPW)], idx1_v, sem1)
    l2 = pltpu.async_copy(idx2_hbm.at[pl.ds(e0, RPW)], idx2_v, sem2)
    l1.wait()
    l2.wait()
    for i in range(RPW // 16):
        sl = pl.ds(i * 16, 16)
        idx1_v[sl] = idx1_v[sl] + boff
        idx2_v[sl] = idx2_v[sl] + boff
    g1 = pltpu.async_copy(table_hbm.at[idx1_v], rows1_v, sem1)
    g2 = pltpu.async_copy(table_hbm.at[idx2_v], rows2_v, sem2)
    g1.wait()
    w1 = pltpu.async_copy(rows1_v, out_s.at[pl.ds(r0, RPW)], sem1)
    g2.wait()
    w2 = pltpu.async_copy(rows2_v, out_r.at[pl.ds(r0, RPW)], sem2)
    w1.wait()
    w2.wait()


def _tc_kernel(ss_ref, sr_ref, bonds_ref, oh2_ref, sites_ref,
               W_self_ref, W_pool_ref, b_eq_ref, att_W_ref, att_b_ref,
               W1_ref, b1_ref, W2_ref, b2_ref, out_ref):
    W_eff = W_self_ref[...] + W_pool_ref[...] * (1.0 / N)
    ss = ss_ref[...][:, :IN_F]
    sr = sr_ref[...][:, :IN_F]
    lat = (jnp.dot(ss, W_eff[:IN_F], preferred_element_type=jnp.float32)
           + jnp.dot(sr, W_eff[IN_F:2 * IN_F], preferred_element_type=jnp.float32)
           + jnp.dot(bonds_ref[...], W_eff[2 * IN_F:], preferred_element_type=jnp.float32)
           + b_eq_ref[...])
    lat = _leaky(lat)
    logits = jnp.sum(lat * att_W_ref[...].T, axis=1, keepdims=True) + att_b_ref[...]
    lat = lat * jax.nn.sigmoid(logits)

    oh2 = oh2_ref[...]
    sites = sites_ref[...]
    h1 = jnp.dot(sites, W1_ref[:IN_F], preferred_element_type=jnp.float32)
    msgs = []
    for b in range(B):
        msgs.append(jnp.dot(oh2.T, lat[b * E:(b + 1) * E],
                            preferred_element_type=jnp.float32))
    msg = jnp.concatenate(msgs, axis=0)

    v = _leaky(h1 + jnp.dot(msg, W1_ref[IN_F:], preferred_element_type=jnp.float32)
               + b1_ref[...])
    v = _leaky(jnp.dot(v, W2_ref[...], preferred_element_type=jnp.float32)
               + b2_ref[...])
    out_ref[...] = sites + v


def kernel(sites, bonds, idx1, idx2, idx2_oh, W_self, W_pool, b_eq, att_W, att_b, W1, b1, W2, b2):
    C = 2 * IN_F + BOND_F
    sites_flat = sites.reshape(B * N, IN_F)
    bonds_flat = bonds.reshape(B * E, BOND_F)
    table = jnp.concatenate(
        [sites_flat, jnp.zeros((B * N, PAD - IN_F), jnp.float32)], axis=1)

    ss, sr = _sc_gather(table, idx1, idx2)

    fixed2 = lambda: (0, 0)
    sites_out = pl.pallas_call(
        _tc_kernel,
        in_specs=[pl.BlockSpec((ROWS, PAD), fixed2),
                  pl.BlockSpec((ROWS, PAD), fixed2),
                  pl.BlockSpec((ROWS, BOND_F), fixed2),
                  pl.BlockSpec((E, N), fixed2),
                  pl.BlockSpec((B * N, IN_F), fixed2),
                  pl.BlockSpec((C, MSG_F), fixed2),
                  pl.BlockSpec((C, MSG_F), fixed2),
                  pl.BlockSpec((1, MSG_F), fixed2),
                  pl.BlockSpec((MSG_F, 1), fixed2),
                  pl.BlockSpec((1, 1), fixed2),
                  pl.BlockSpec((IN_F + MSG_F, HID_F), fixed2),
                  pl.BlockSpec((1, HID_F), fixed2),
                  pl.BlockSpec((HID_F, OUT_F), fixed2),
                  pl.BlockSpec((1, OUT_F), fixed2)],
        out_specs=pl.BlockSpec((B * N, OUT_F), fixed2),
        out_shape=jax.ShapeDtypeStruct((B * N, OUT_F), jnp.float32),
    )(ss, sr, bonds_flat, idx2_oh, sites_flat, W_self, W_pool,
      b_eq.reshape(1, MSG_F), att_W, att_b.reshape(1, 1),
      W1, b1.reshape(1, HID_F), W2, b2.reshape(1, OUT_F))

    return (sites_out.reshape(B, N, OUT_F), bonds)
